# baseline (speedup 1.0000x reference)
import functools

import jax
import jax.numpy as jnp
from jax import lax
from jax.experimental import pallas as pl
from jax.experimental.pallas import tpu as pltpu
from jax.experimental.pallas import tpu_sc as plsc

_PAD = 1
_L = 16
_NC = 2
_NS = 16
_NW = _NC * _NS

_CHUNK = 32
_NBUF = 3


def _make_sc_kernel(B, Lseq, V, D):
    seg = (B * Lseq) // _NW
    segs_per_row = Lseq // seg
    n_chunks = seg // _CHUNK
    mesh = plsc.VectorSubcoreMesh(core_axis_name="c", subcore_axis_name="s")

    @functools.partial(
        pl.kernel,
        mesh=mesh,
        compiler_params=pltpu.CompilerParams(needs_layout_passes=False),
        out_type=jax.ShapeDtypeStruct((B * Lseq, D), jnp.float32),
        scratch_types=[
            pltpu.VMEM((Lseq,), jnp.int32),
            pltpu.VMEM((seg,), jnp.int32),
            pltpu.VMEM((_NBUF, _CHUNK, D), jnp.float32),
        ] + [pltpu.VMEM((_CHUNK,), jnp.int32) for _ in range(n_chunks)]
          + [pltpu.VMEM((_NBUF, _CHUNK), jnp.int32)]
          + [pltpu.SemaphoreType.DMA] * (2 * _NBUF + 2),
    )
    def sc_kernel(pos_hbm, weight_hbm, out_hbm, row_v, seg_v, bufs,
                  *rest):
        idx_refs = rest[:n_chunks]
        spec_ref = rest[n_chunks]
        sems = rest[n_chunks + 1:]
        gsems = sems[:_NBUF]
        wsems = sems[_NBUF:2 * _NBUF]
        s0, s1 = sems[2 * _NBUF], sems[2 * _NBUF + 1]
        wid = lax.axis_index("s") * _NC + lax.axis_index("c")
        b = wid // segs_per_row
        s = wid % segs_per_row
        out_base = wid * seg
        lane = lax.broadcasted_iota(jnp.int32, (_L,), 0)

        gathers = [None] * n_chunks
        for c in range(_NBUF):
            base = s * seg + c * _CHUNK + 2
            for kk in range(_CHUNK // _L):
                spec_ref[c, pl.ds(kk * _L, _L)] = lane + (base + kk * _L)
            gathers[c] = pltpu.async_copy(
                weight_hbm.at[spec_ref.at[c]], bufs.at[c], gsems[c])

        c_row = pltpu.async_copy(pos_hbm.at[b], row_v, s0)
        c_seg = pltpu.async_copy(pos_hbm.at[b, pl.ds(s * seg, seg)], seg_v, s1)
        c_seg.wait()
        c_row.wait()

        s_chunks = s * (seg // _L)
        acc = jnp.zeros((_L,), jnp.int32)
        for j in range((segs_per_row - 1) * (seg // _L)):
            v = row_v[pl.ds(j * _L, _L)]
            pad = jnp.where(v != _PAD, 1, 0)
            gate = jnp.where(j < s_chunks, 1, 0)
            acc = acc + pad * gate
        offset = jnp.sum(acc)

        writes = [None] * n_chunks
        carry = offset
        for c in range(n_chunks):
            mism = jnp.zeros((), jnp.int32)
            for kk in range(_CHUNK // _L):
                k = c * (_CHUNK // _L) + kk
                v = seg_v[pl.ds(k * _L, _L)]
                m = v != _PAD
                mi = jnp.where(m, 1, 0)
                cs = plsc.cumsum(mi)
                pos = jnp.where(m, cs + carry, 0) + _PAD
                carry = carry + jnp.sum(mi)
                idx_refs[c][pl.ds(kk * _L, _L)] = pos
                if c < _NBUF:
                    spec = lane + (s * seg + c * _CHUNK + kk * _L + 2)
                    mism = mism + jnp.sum(jnp.where(pos != spec, 1, 0))
            bb = c % _NBUF
            if c < _NBUF:
                gathers[c].wait()

                @pl.when(mism != 0)
                def _fixup(c=c, bb=bb):
                    pltpu.async_copy(
                        weight_hbm.at[idx_refs[c]], bufs.at[bb],
                        gsems[bb]).wait()
                writes[c] = pltpu.async_copy(
                    bufs.at[bb],
                    out_hbm.at[pl.ds(out_base + c * _CHUNK, _CHUNK)],
                    wsems[bb])
            else:
                writes[c - _NBUF].wait()
                gathers[c] = pltpu.async_copy(
                    weight_hbm.at[idx_refs[c]], bufs.at[bb], gsems[bb])
                if c - 1 >= _NBUF:
                    gathers[c - 1].wait()
                    writes[c - 1] = pltpu.async_copy(
                        bufs.at[(c - 1) % _NBUF],
                        out_hbm.at[pl.ds(out_base + (c - 1) * _CHUNK, _CHUNK)],
                        wsems[(c - 1) % _NBUF])
        last = n_chunks - 1
        gathers[last].wait()
        writes[last] = pltpu.async_copy(
            bufs.at[last % _NBUF],
            out_hbm.at[pl.ds(out_base + last * _CHUNK, _CHUNK)],
            wsems[last % _NBUF])
        for c in range(n_chunks - _NBUF, n_chunks):
            writes[c].wait()

    return sc_kernel


def kernel(position_ids, weight):
    B, Lseq = position_ids.shape
    V, D = weight.shape
    sc = _make_sc_kernel(B, Lseq, V, D)
    out = sc(position_ids, weight)
    return out.reshape(B, Lseq, D)


# speedup vs baseline: 1.0853x; 1.0853x over previous
import functools

import jax
import jax.numpy as jnp
from jax import lax
from jax.experimental import pallas as pl
from jax.experimental.pallas import tpu as pltpu
from jax.experimental.pallas import tpu_sc as plsc

_PAD = 1
_L = 16
_NC = 2
_NS = 16
_NW = _NC * _NS

_CHUNK = 32
_NBUF = 3


def _make_sc_kernel(B, Lseq, V, D):
    seg = (B * Lseq) // _NW
    segs_per_row = Lseq // seg
    n_chunks = seg // _CHUNK
    mesh = plsc.VectorSubcoreMesh(core_axis_name="c", subcore_axis_name="s")

    @functools.partial(
        pl.kernel,
        mesh=mesh,
        compiler_params=pltpu.CompilerParams(needs_layout_passes=False),
        out_type=jax.ShapeDtypeStruct((B * Lseq, D), jnp.float32),
        scratch_types=[
            pltpu.VMEM((Lseq,), jnp.int32),
            pltpu.VMEM((seg,), jnp.int32),
            pltpu.VMEM((_NBUF, _CHUNK, D), jnp.float32),
        ] + [pltpu.VMEM((_CHUNK,), jnp.int32) for _ in range(n_chunks)]
          + [pltpu.SemaphoreType.DMA] * (2 * _NBUF + 2),
    )
    def sc_kernel(pos_hbm, weight_hbm, out_hbm, row_v, seg_v, bufs,
                  *rest):
        idx_refs = rest[:n_chunks]
        sems = rest[n_chunks:]
        gsems = sems[:_NBUF]
        wsems = sems[_NBUF:2 * _NBUF]
        s0, s1 = sems[2 * _NBUF], sems[2 * _NBUF + 1]
        wid = lax.axis_index("s") * _NC + lax.axis_index("c")
        b = wid // segs_per_row
        s = wid % segs_per_row
        out_base = wid * seg

        c_row = pltpu.async_copy(pos_hbm.at[b], row_v, s0)
        c_seg = pltpu.async_copy(pos_hbm.at[b, pl.ds(s * seg, seg)], seg_v, s1)
        c_seg.wait()
        c_row.wait()

        s_chunks = s * (seg // _L)
        acc = jnp.zeros((_L,), jnp.int32)
        for j in range((segs_per_row - 1) * (seg // _L)):
            v = row_v[pl.ds(j * _L, _L)]
            pad = jnp.where(v != _PAD, 1, 0)
            gate = jnp.where(j < s_chunks, 1, 0)
            acc = acc + pad * gate
        offset = jnp.sum(acc)

        gathers = [None] * n_chunks
        writes = [None] * n_chunks
        carry = offset
        for c in range(n_chunks):
            for kk in range(_CHUNK // _L):
                k = c * (_CHUNK // _L) + kk
                v = seg_v[pl.ds(k * _L, _L)]
                m = v != _PAD
                mi = jnp.where(m, 1, 0)
                cs = plsc.cumsum(mi)
                pos = jnp.where(m, cs + carry, 0) + _PAD
                carry = carry + jnp.sum(mi)
                idx_refs[c][pl.ds(kk * _L, _L)] = pos
            bb = c % _NBUF
            if c >= _NBUF:
                writes[c - _NBUF].wait()
            gathers[c] = pltpu.async_copy(
                weight_hbm.at[idx_refs[c]], bufs.at[bb], gsems[bb])
            if c >= 1:
                gathers[c - 1].wait()
                writes[c - 1] = pltpu.async_copy(
                    bufs.at[(c - 1) % _NBUF],
                    out_hbm.at[pl.ds(out_base + (c - 1) * _CHUNK, _CHUNK)],
                    wsems[(c - 1) % _NBUF])
        last = n_chunks - 1
        gathers[last].wait()
        writes[last] = pltpu.async_copy(
            bufs.at[last % _NBUF],
            out_hbm.at[pl.ds(out_base + last * _CHUNK, _CHUNK)],
            wsems[last % _NBUF])
        for c in range(n_chunks - _NBUF, n_chunks):
            writes[c].wait()

    return sc_kernel


def kernel(position_ids, weight):
    B, Lseq = position_ids.shape
    V, D = weight.shape
    sc = _make_sc_kernel(B, Lseq, V, D)
    out = sc(position_ids, weight)
    return out.reshape(B, Lseq, D)
